# Initial kernel scaffold; baseline (speedup 1.0000x reference)
import jax
import jax.numpy as jnp
from jax.experimental import pallas as pl


def kernel(x, edge_index, edge_attr, W1, b1, W2, b2, W3, b3):
    raise NotImplementedError("write your pallas kernel here")


# speedup vs baseline: 32.1203x; 32.1203x over previous
import functools
import jax
import jax.numpy as jnp
from jax import lax
from jax.experimental import pallas as pl
from jax.experimental.pallas import tpu as pltpu
from jax.experimental.pallas import tpu_sc as plsc

N = 10000
E = 320000
D = 128
B = 2
EH = 32
EO = 30
EOP = 32
NOUT = 128

NC = 2
NS = 16
NW = NC * NS
CH = 128
CPW = (B * E + NW * CH - 1) // (NW * CH)
EPW = CPW * CH
EP = NW * EPW
BN = B * N
ACC_R = BN + NS
RPT = ACC_R // NS

@functools.lru_cache(maxsize=None)
def _sc_mesh():
    return plsc.VectorSubcoreMesh(
        core_axis_name="c", subcore_axis_name="s",
        num_cores=NC, num_subcores=NS)


def _stats_body(ea_ref, o_ref):
    ea = ea_ref[...]
    s = jnp.sum(ea)
    ss = jnp.sum(ea * ea)
    mean = s / E
    var = (ss - s * s / E) / (E - 1)
    alpha = lax.rsqrt(var)
    o_ref[0] = alpha
    o_ref[1] = -mean * alpha


def _stats_call(ea):
    return pl.pallas_call(
        _stats_body,
        out_specs=pl.BlockSpec(memory_space=pltpu.SMEM),
        out_shape=jax.ShapeDtypeStruct((2,), jnp.float32),
    )(ea)


_TBLK = 2000


def _tables_body(x_ref, wa_ref, wb_ref, b1_ref, hs_ref, hd_ref):
    xb = x_ref[...]
    hs_ref[...] = jnp.dot(xb, wa_ref[...],
                          preferred_element_type=jnp.float32) + b1_ref[...]
    hd_ref[...] = jnp.dot(xb, wb_ref[...],
                          preferred_element_type=jnp.float32)


def _tables_call(x2, W1a, W1b, b1r):
    g = BN // _TBLK
    return pl.pallas_call(
        _tables_body,
        grid=(g,),
        in_specs=[
            pl.BlockSpec((_TBLK, D), lambda i: (i, 0)),
            pl.BlockSpec((D, EH), lambda i: (0, 0)),
            pl.BlockSpec((D, EH), lambda i: (0, 0)),
            pl.BlockSpec((1, EH), lambda i: (0, 0)),
        ],
        out_specs=[
            pl.BlockSpec((_TBLK, EH), lambda i: (i, 0)),
            pl.BlockSpec((_TBLK, EH), lambda i: (i, 0)),
        ],
        out_shape=[
            jax.ShapeDtypeStruct((BN, EH), jnp.float32),
            jax.ShapeDtypeStruct((BN, EH), jnp.float32),
        ],
    )(x2, W1a, W1b, b1r)


def _gather_body(srcg, dstg, hs, hd, gs, gd, idx_s, idx_d, buf_s, buf_d,
                 sem_s, sem_d):
    c = lax.axis_index("c")
    s = lax.axis_index("s")
    w = s * NC + c
    pltpu.sync_copy(srcg.at[w], idx_s)
    pltpu.sync_copy(dstg.at[w], idx_d)

    def step(j, carry):
        cp_s = pltpu.async_copy(hs.at[idx_s.at[j]], buf_s, sem_s)
        cp_d = pltpu.async_copy(hd.at[idx_d.at[j]], buf_d, sem_d)
        cp_s.wait()
        cp_d.wait()
        base = w * EPW + j * CH
        pltpu.sync_copy(buf_s, gs.at[pl.ds(base, CH)])
        pltpu.sync_copy(buf_d, gd.at[pl.ds(base, CH)])
        return carry

    lax.fori_loop(0, CPW, step, 0)


@functools.lru_cache(maxsize=None)
def _gather_kernel():
    return pl.kernel(
        _gather_body,
        out_type=[
            jax.ShapeDtypeStruct((EP, EH), jnp.float32),
            jax.ShapeDtypeStruct((EP, EH), jnp.float32),
        ],
        mesh=_sc_mesh(),
        compiler_params=pltpu.CompilerParams(use_tc_tiling_on_sc=False),
        scratch_types=[
            pltpu.VMEM((CPW, CH), jnp.int32),
            pltpu.VMEM((CPW, CH), jnp.int32),
            pltpu.VMEM((CH, EH), jnp.float32),
            pltpu.VMEM((CH, EH), jnp.float32),
            pltpu.SemaphoreType.DMA,
            pltpu.SemaphoreType.DMA,
        ],
    )


_EBLK = 2048


def _edge_body(gs_ref, gd_ref, ea_ref, st_ref, w1c_ref, W2_ref, b2_ref,
               u_ref, un_ref):
    alpha = st_ref[0]
    beta = st_ref[1]
    c = ea_ref[...] * alpha + beta
    z = jax.nn.sigmoid(gs_ref[...] + gd_ref[...] + c * w1c_ref[...])
    u = jax.nn.sigmoid(
        jnp.dot(z, W2_ref[...], preferred_element_type=jnp.float32)
        + b2_ref[...])
    u_ref[...] = u
    un_ref[...] = -u


def _edge_call(gs, gd, ea2, stats, w1c, W2p, b2p):
    g = EP // _EBLK
    return pl.pallas_call(
        _edge_body,
        grid=(g,),
        in_specs=[
            pl.BlockSpec((_EBLK, EH), lambda i: (i, 0)),
            pl.BlockSpec((_EBLK, EH), lambda i: (i, 0)),
            pl.BlockSpec((_EBLK, 1), lambda i: (i, 0)),
            pl.BlockSpec(memory_space=pltpu.SMEM),
            pl.BlockSpec((1, EH), lambda i: (0, 0)),
            pl.BlockSpec((EH, EOP), lambda i: (0, 0)),
            pl.BlockSpec((1, EOP), lambda i: (0, 0)),
        ],
        out_specs=[
            pl.BlockSpec((_EBLK, EOP), lambda i: (i, 0)),
            pl.BlockSpec((_EBLK, EOP), lambda i: (i, 0)),
        ],
        out_shape=[
            jax.ShapeDtypeStruct((EP, EOP), jnp.float32),
            jax.ShapeDtypeStruct((EP, EOP), jnp.float32),
        ],
    )(gs, gd, ea2, stats, w1c, W2p, b2p)


def _scatter_body(srcs, dsts, u, un, acc_out, idx_s, idx_d, ubuf, nbuf, zbuf,
                  acc_p, sem):
    c = lax.axis_index("c")
    s = lax.axis_index("s")
    w = s * NC + c

    zv = jnp.zeros((16,), jnp.float32)

    def zstep(i, carry):
        zbuf[i, pl.ds(0, 16)] = zv
        zbuf[i, pl.ds(16, 16)] = zv
        return carry

    lax.fori_loop(0, RPT, zstep, 0)
    pltpu.sync_copy(zbuf, acc_p.at[pl.ds(s * RPT, RPT)])
    pltpu.sync_copy(srcs.at[w], idx_s)
    pltpu.sync_copy(dsts.at[w], idx_d)
    plsc.subcore_barrier()

    def step(j, carry):
        pltpu.sync_copy(u.at[pl.ds(w * EPW + j * CH, CH)], ubuf)
        pltpu.sync_copy(ubuf, acc_p.at[idx_d.at[j]], add=True)
        pltpu.sync_copy(un.at[pl.ds(w * EPW + j * CH, CH)], nbuf)
        pltpu.sync_copy(nbuf, acc_p.at[idx_s.at[j]], add=True)
        return carry

    lax.fori_loop(0, CPW, step, 0)
    plsc.subcore_barrier()
    pltpu.sync_copy(acc_p.at[pl.ds(s * RPT, RPT)],
                    acc_out.at[c, pl.ds(s * RPT, RPT)])


@functools.lru_cache(maxsize=None)
def _scatter_kernel():
    return pl.kernel(
        _scatter_body,
        out_type=jax.ShapeDtypeStruct((NC, ACC_R, EOP), jnp.float32),
        mesh=_sc_mesh(),
        compiler_params=pltpu.CompilerParams(use_tc_tiling_on_sc=False),
        scratch_types=[
            pltpu.VMEM((CPW, CH), jnp.int32),
            pltpu.VMEM((CPW, CH), jnp.int32),
            pltpu.VMEM((CH, EOP), jnp.float32),
            pltpu.VMEM((CH, EOP), jnp.float32),
            pltpu.VMEM((RPT, EOP), jnp.float32),
            pltpu.VMEM_SHARED((ACC_R, EOP), jnp.float32),
            pltpu.SemaphoreType.DMA,
        ],
    )


_NBLK = 2000


def _final_body(p0_ref, p1_ref, W3_ref, b3_ref, o_ref):
    agg = p0_ref[...] + p1_ref[...]
    o_ref[...] = jax.nn.sigmoid(
        jnp.dot(agg, W3_ref[...], preferred_element_type=jnp.float32)
        + b3_ref[...])


def _final_call(p0, p1, W3p, b3r):
    g = BN // _NBLK
    return pl.pallas_call(
        _final_body,
        grid=(g,),
        in_specs=[
            pl.BlockSpec((_NBLK, EOP), lambda i: (i, 0)),
            pl.BlockSpec((_NBLK, EOP), lambda i: (i, 0)),
            pl.BlockSpec((EOP, NOUT), lambda i: (0, 0)),
            pl.BlockSpec((1, NOUT), lambda i: (0, 0)),
        ],
        out_specs=pl.BlockSpec((_NBLK, NOUT), lambda i: (i, 0)),
        out_shape=jax.ShapeDtypeStruct((BN, NOUT), jnp.float32),
    )(p0, p1, W3p, b3r)


def kernel(x, edge_index, edge_attr, W1, b1, W2, b2, W3, b3):
    f32 = jnp.float32
    x2 = x.reshape(BN, D).astype(f32)
    ea = edge_attr.astype(f32)
    src = edge_index[0].astype(jnp.int32)
    dst = edge_index[1].astype(jnp.int32)

    W1 = W1.astype(f32)
    W1a = W1[:D]
    W1b = W1[D:2 * D]
    w1c = W1[2 * D].reshape(1, EH)
    b1r = b1.astype(f32).reshape(1, EH)
    W2p = jnp.pad(W2.astype(f32), ((0, 0), (0, EOP - EO)))
    b2p = jnp.pad(b2.astype(f32), (0, EOP - EO)).reshape(1, EOP)
    W3p = jnp.pad(W3.astype(f32), ((0, EOP - EO), (0, 0)))
    b3r = b3.astype(f32).reshape(1, NOUT)

    npad = EP - B * E
    srcb = jnp.concatenate([src, src + N])
    dstb = jnp.concatenate([dst, dst + N])
    zpad = jnp.zeros((npad,), jnp.int32)
    dpad = jnp.full((npad,), BN, jnp.int32)
    srcg = jnp.concatenate([srcb, zpad]).reshape(NW, CPW, CH)
    dstg = jnp.concatenate([dstb, zpad]).reshape(NW, CPW, CH)
    srcs = jnp.concatenate([srcb, dpad]).reshape(NW, CPW, CH)
    dsts = jnp.concatenate([dstb, dpad]).reshape(NW, CPW, CH)
    ea2 = jnp.concatenate([ea, ea, jnp.zeros((npad, 1), f32)])

    stats = _stats_call(ea.reshape(E // 128, 128))
    hs, hd = _tables_call(x2, W1a, W1b, b1r)
    gs, gd = _gather_kernel()(srcg, dstg, hs, hd)
    u, un = _edge_call(gs, gd, ea2, stats, w1c, W2p, b2p)
    acc = _scatter_kernel()(srcs, dsts, u, un)
    out = _final_call(acc[0, :BN], acc[1, :BN], W3p, b3r)
    return out.reshape(B, N, NOUT)


# speedup vs baseline: 78.2077x; 2.4348x over previous
import functools
import jax
import jax.numpy as jnp
from jax import lax
from jax.experimental import pallas as pl
from jax.experimental.pallas import tpu as pltpu
from jax.experimental.pallas import tpu_sc as plsc

N = 10000
E = 320000
D = 128
B = 2
EH = 32
EO = 30
EOP = 32
NOUT = 128

NC = 2
NS = 16
NW = NC * NS
CH = 128
CPW = (B * E + NW * CH - 1) // (NW * CH)
EPW = CPW * CH
EP = NW * EPW
BN = B * N
ACC_R = BN + NS
RPT = ACC_R // NS

@functools.lru_cache(maxsize=None)
def _sc_mesh():
    return plsc.VectorSubcoreMesh(
        core_axis_name="c", subcore_axis_name="s",
        num_cores=NC, num_subcores=NS)


def _stats_body(ea_ref, o_ref):
    ea = ea_ref[...]
    s = jnp.sum(ea)
    ss = jnp.sum(ea * ea)
    mean = s / E
    var = (ss - s * s / E) / (E - 1)
    alpha = lax.rsqrt(var)
    o_ref[0] = alpha
    o_ref[1] = -mean * alpha


def _stats_call(ea):
    return pl.pallas_call(
        _stats_body,
        out_specs=pl.BlockSpec(memory_space=pltpu.SMEM),
        out_shape=jax.ShapeDtypeStruct((2,), jnp.float32),
    )(ea)


_TBLK = 2000


def _tables_body(x_ref, wa_ref, wb_ref, b1_ref, hs_ref, hd_ref):
    xb = x_ref[...]
    hs_ref[...] = jnp.dot(xb, wa_ref[...],
                          preferred_element_type=jnp.float32) + b1_ref[...]
    hd_ref[...] = jnp.dot(xb, wb_ref[...],
                          preferred_element_type=jnp.float32)


def _tables_call(x2, W1a, W1b, b1r):
    g = BN // _TBLK
    return pl.pallas_call(
        _tables_body,
        grid=(g,),
        in_specs=[
            pl.BlockSpec((_TBLK, D), lambda i: (i, 0)),
            pl.BlockSpec((D, EH), lambda i: (0, 0)),
            pl.BlockSpec((D, EH), lambda i: (0, 0)),
            pl.BlockSpec((1, EH), lambda i: (0, 0)),
        ],
        out_specs=[
            pl.BlockSpec((_TBLK, EH), lambda i: (i, 0)),
            pl.BlockSpec((_TBLK, EH), lambda i: (i, 0)),
        ],
        out_shape=[
            jax.ShapeDtypeStruct((BN, EH), jnp.float32),
            jax.ShapeDtypeStruct((BN, EH), jnp.float32),
        ],
    )(x2, W1a, W1b, b1r)


def _gather_body(srcg, dstg, hs, hd, gs, gd, idx_s, idx_d, buf_s, buf_d,
                 sem_s, sem_d):
    c = lax.axis_index("c")
    s = lax.axis_index("s")
    w = s * NC + c
    pltpu.sync_copy(srcg.at[w], idx_s)
    pltpu.sync_copy(dstg.at[w], idx_d)

    def step(j, carry):
        cp_s = pltpu.async_copy(hs.at[idx_s.at[j]], buf_s, sem_s)
        cp_d = pltpu.async_copy(hd.at[idx_d.at[j]], buf_d, sem_d)
        cp_s.wait()
        cp_d.wait()
        base = w * EPW + j * CH
        pltpu.sync_copy(buf_s, gs.at[pl.ds(base, CH)])
        pltpu.sync_copy(buf_d, gd.at[pl.ds(base, CH)])
        return carry

    lax.fori_loop(0, CPW, step, 0)


@functools.lru_cache(maxsize=None)
def _gather_kernel():
    return pl.kernel(
        _gather_body,
        out_type=[
            jax.ShapeDtypeStruct((EP, EH), jnp.float32),
            jax.ShapeDtypeStruct((EP, EH), jnp.float32),
        ],
        mesh=_sc_mesh(),
        compiler_params=pltpu.CompilerParams(use_tc_tiling_on_sc=False),
        scratch_types=[
            pltpu.VMEM((CPW, CH), jnp.int32),
            pltpu.VMEM((CPW, CH), jnp.int32),
            pltpu.VMEM((CH, EH), jnp.float32),
            pltpu.VMEM((CH, EH), jnp.float32),
            pltpu.SemaphoreType.DMA,
            pltpu.SemaphoreType.DMA,
        ],
    )


_EBLK = 1024
EP4 = EP // 4


def _edge_body(gs_ref, gd_ref, ea_ref, st_ref, SW_ref, BD2_ref, b2_ref,
               u_ref, un_ref):
    alpha = st_ref[0]
    beta = st_ref[1]
    c4t = ea_ref[...] * alpha + beta
    att = lax.dot_general(c4t, SW_ref[...], (((0,), (0,)), ((), ())),
                          preferred_element_type=jnp.float32)
    z = jax.nn.sigmoid(gs_ref[...] + gd_ref[...] + att)
    u = jax.nn.sigmoid(
        jnp.dot(z, BD2_ref[...], preferred_element_type=jnp.float32)
        + b2_ref[...])
    u_ref[...] = u
    un_ref[...] = -u


def _edge_call(gs4, gd4, ea4, stats, SW, BD2, b2t):
    g = EP4 // _EBLK
    return pl.pallas_call(
        _edge_body,
        grid=(g,),
        in_specs=[
            pl.BlockSpec((_EBLK, 128), lambda i: (i, 0)),
            pl.BlockSpec((_EBLK, 128), lambda i: (i, 0)),
            pl.BlockSpec((4, _EBLK), lambda i: (0, i)),
            pl.BlockSpec(memory_space=pltpu.SMEM),
            pl.BlockSpec((4, 128), lambda i: (0, 0)),
            pl.BlockSpec((128, 128), lambda i: (0, 0)),
            pl.BlockSpec((1, 128), lambda i: (0, 0)),
        ],
        out_specs=[
            pl.BlockSpec((_EBLK, 128), lambda i: (i, 0)),
            pl.BlockSpec((_EBLK, 128), lambda i: (i, 0)),
        ],
        out_shape=[
            jax.ShapeDtypeStruct((EP4, 128), jnp.float32),
            jax.ShapeDtypeStruct((EP4, 128), jnp.float32),
        ],
    )(gs4, gd4, ea4, stats, SW, BD2, b2t)


def _scatter_body(srcs, dsts, u, un, acc_out, idx_s, idx_d, ubuf, nbuf, zbuf,
                  acc_p, sem):
    c = lax.axis_index("c")
    s = lax.axis_index("s")
    w = s * NC + c

    zv = jnp.zeros((16,), jnp.float32)

    def zstep(i, carry):
        zbuf[i, pl.ds(0, 16)] = zv
        zbuf[i, pl.ds(16, 16)] = zv
        return carry

    lax.fori_loop(0, RPT, zstep, 0)
    pltpu.sync_copy(zbuf, acc_p.at[pl.ds(s * RPT, RPT)])
    pltpu.sync_copy(srcs.at[w], idx_s)
    pltpu.sync_copy(dsts.at[w], idx_d)
    plsc.subcore_barrier()

    def step(j, carry):
        pltpu.sync_copy(u.at[pl.ds(w * EPW + j * CH, CH)], ubuf)
        pltpu.sync_copy(ubuf, acc_p.at[idx_d.at[j]], add=True)
        pltpu.sync_copy(un.at[pl.ds(w * EPW + j * CH, CH)], nbuf)
        pltpu.sync_copy(nbuf, acc_p.at[idx_s.at[j]], add=True)
        return carry

    lax.fori_loop(0, CPW, step, 0)
    plsc.subcore_barrier()
    pltpu.sync_copy(acc_p.at[pl.ds(s * RPT, RPT)],
                    acc_out.at[c, pl.ds(s * RPT, RPT)])


@functools.lru_cache(maxsize=None)
def _scatter_kernel():
    return pl.kernel(
        _scatter_body,
        out_type=jax.ShapeDtypeStruct((NC, ACC_R, EOP), jnp.float32),
        mesh=_sc_mesh(),
        compiler_params=pltpu.CompilerParams(use_tc_tiling_on_sc=False),
        scratch_types=[
            pltpu.VMEM((CPW, CH), jnp.int32),
            pltpu.VMEM((CPW, CH), jnp.int32),
            pltpu.VMEM((CH, EOP), jnp.float32),
            pltpu.VMEM((CH, EOP), jnp.float32),
            pltpu.VMEM((RPT, EOP), jnp.float32),
            pltpu.VMEM_SHARED((ACC_R, EOP), jnp.float32),
            pltpu.SemaphoreType.DMA,
        ],
    )


_NBLK = 2000


def _final_body(p0_ref, p1_ref, W3_ref, b3_ref, o_ref):
    agg = p0_ref[...] + p1_ref[...]
    o_ref[...] = jax.nn.sigmoid(
        jnp.dot(agg, W3_ref[...], preferred_element_type=jnp.float32)
        + b3_ref[...])


def _final_call(p0, p1, W3p, b3r):
    g = BN // _NBLK
    return pl.pallas_call(
        _final_body,
        grid=(g,),
        in_specs=[
            pl.BlockSpec((_NBLK, EOP), lambda i: (i, 0)),
            pl.BlockSpec((_NBLK, EOP), lambda i: (i, 0)),
            pl.BlockSpec((EOP, NOUT), lambda i: (0, 0)),
            pl.BlockSpec((1, NOUT), lambda i: (0, 0)),
        ],
        out_specs=pl.BlockSpec((_NBLK, NOUT), lambda i: (i, 0)),
        out_shape=jax.ShapeDtypeStruct((BN, NOUT), jnp.float32),
    )(p0, p1, W3p, b3r)


def kernel(x, edge_index, edge_attr, W1, b1, W2, b2, W3, b3):
    f32 = jnp.float32
    x2 = x.reshape(BN, D).astype(f32)
    ea = edge_attr.astype(f32)
    src = edge_index[0].astype(jnp.int32)
    dst = edge_index[1].astype(jnp.int32)

    W1 = W1.astype(f32)
    W1a = W1[:D]
    W1b = W1[D:2 * D]
    w1c = W1[2 * D].reshape(1, EH)
    b1r = b1.astype(f32).reshape(1, EH)
    W2p = jnp.pad(W2.astype(f32), ((0, 0), (0, EOP - EO)))
    b2p = jnp.pad(b2.astype(f32), (0, EOP - EO)).reshape(1, EOP)
    W3p = jnp.pad(W3.astype(f32), ((0, EOP - EO), (0, 0)))
    b3r = b3.astype(f32).reshape(1, NOUT)

    npad = EP - B * E
    srcb = jnp.concatenate([src, src + N])
    dstb = jnp.concatenate([dst, dst + N])
    zpad = jnp.zeros((npad,), jnp.int32)
    dpad = jnp.full((npad,), BN, jnp.int32)
    srcg = jnp.concatenate([srcb, zpad]).reshape(NW, CPW, CH)
    dstg = jnp.concatenate([dstb, zpad]).reshape(NW, CPW, CH)
    srcs = jnp.concatenate([srcb, dpad]).reshape(NW, CPW, CH)
    dsts = jnp.concatenate([dstb, dpad]).reshape(NW, CPW, CH)
    ea2 = jnp.concatenate([ea, ea, jnp.zeros((npad, 1), f32)])

    eye4 = jnp.eye(4, dtype=f32)
    SW = jnp.kron(eye4, w1c)
    BD2 = jnp.kron(eye4, W2p)
    b2t = jnp.tile(b2p, (1, 4))
    eaf = ea2.reshape(EP)
    ea4 = jnp.stack([eaf[0::4], eaf[1::4], eaf[2::4], eaf[3::4]])

    stats = _stats_call(ea.reshape(E // 128, 128))
    hs, hd = _tables_call(x2, W1a, W1b, b1r)
    gs, gd = _gather_kernel()(srcg, dstg, hs, hd)
    gs4 = gs.reshape(EP // 4, 128)
    gd4 = gd.reshape(EP // 4, 128)
    u4, un4 = _edge_call(gs4, gd4, ea4, stats, SW, BD2, b2t)
    acc = _scatter_kernel()(srcs, dsts, u4.reshape(EP, EOP),
                            un4.reshape(EP, EOP))
    out = _final_call(acc[0, :BN], acc[1, :BN], W3p, b3r)
    return out.reshape(B, N, NOUT)


# speedup vs baseline: 112.9485x; 1.4442x over previous
import functools
import jax
import jax.numpy as jnp
from jax import lax
from jax.experimental import pallas as pl
from jax.experimental.pallas import tpu as pltpu
from jax.experimental.pallas import tpu_sc as plsc

N = 10000
E = 320000
D = 128
B = 2
EH = 32
EO = 30
EOP = 32
NOUT = 128

NC = 2
NS = 16
NW = NC * NS
CH = 128
CPW = (B * E + NW * CH - 1) // (NW * CH)
EPW = CPW * CH
EP = NW * EPW
BN = B * N
ACC_R = BN + NS
RPT = ACC_R // NS

@functools.lru_cache(maxsize=None)
def _sc_mesh():
    return plsc.VectorSubcoreMesh(
        core_axis_name="c", subcore_axis_name="s",
        num_cores=NC, num_subcores=NS)


def _stats_body(ea_ref, o_ref):
    ea = ea_ref[...]
    s = jnp.sum(ea)
    ss = jnp.sum(ea * ea)
    mean = s / E
    var = (ss - s * s / E) / (E - 1)
    alpha = lax.rsqrt(var)
    o_ref[0] = alpha
    o_ref[1] = -mean * alpha


def _stats_call(ea):
    return pl.pallas_call(
        _stats_body,
        out_specs=pl.BlockSpec(memory_space=pltpu.SMEM),
        out_shape=jax.ShapeDtypeStruct((2,), jnp.float32),
    )(ea)


_TBLK = 2000


def _tables_body(x_ref, wa_ref, wb_ref, b1_ref, hs_ref, hd_ref):
    xb = x_ref[...]
    hs_ref[...] = jnp.dot(xb, wa_ref[...],
                          preferred_element_type=jnp.float32) + b1_ref[...]
    hd_ref[...] = jnp.dot(xb, wb_ref[...],
                          preferred_element_type=jnp.float32)


def _tables_call(x2, W1a, W1b, b1r):
    g = BN // _TBLK
    return pl.pallas_call(
        _tables_body,
        grid=(g,),
        in_specs=[
            pl.BlockSpec((_TBLK, D), lambda i: (i, 0)),
            pl.BlockSpec((D, EH), lambda i: (0, 0)),
            pl.BlockSpec((D, EH), lambda i: (0, 0)),
            pl.BlockSpec((1, EH), lambda i: (0, 0)),
        ],
        out_specs=[
            pl.BlockSpec((_TBLK, EH), lambda i: (i, 0)),
            pl.BlockSpec((_TBLK, EH), lambda i: (i, 0)),
        ],
        out_shape=[
            jax.ShapeDtypeStruct((BN, EH), jnp.float32),
            jax.ShapeDtypeStruct((BN, EH), jnp.float32),
        ],
    )(x2, W1a, W1b, b1r)


KF = 8
NPH = CPW // KF


def _gather_body(srcg, dstg, hs, hd, gs, gd, idx_s, idx_d, buf_s, buf_d,
                 sem_s, sem_d, sem_ss, sem_sd):
    c = lax.axis_index("c")
    s = lax.axis_index("s")
    w = s * NC + c
    pltpu.sync_copy(srcg.at[w], idx_s)
    pltpu.sync_copy(dstg.at[w], idx_d)

    def do_chunks(base, nk):
        cps = []
        for k in range(nk):
            j = base + k
            b = pl.ds(k * CH, CH)
            cps.append(pltpu.async_copy(hs.at[idx_s.at[j]], buf_s.at[b],
                                        sem_s))
            cps.append(pltpu.async_copy(hd.at[idx_d.at[j]], buf_d.at[b],
                                        sem_d))
        for cp in cps:
            cp.wait()
        sts = []
        for k in range(nk):
            j = base + k
            b = pl.ds(k * CH, CH)
            o = pl.ds(w * EPW + j * CH, CH)
            sts.append(pltpu.async_copy(buf_s.at[b], gs.at[o], sem_ss))
            sts.append(pltpu.async_copy(buf_d.at[b], gd.at[o], sem_sd))
        for st in sts:
            st.wait()

    def phase(g, carry):
        do_chunks(g * KF, KF)
        return carry

    lax.fori_loop(0, NPH, phase, 0)
    do_chunks(NPH * KF, CPW - NPH * KF)


@functools.lru_cache(maxsize=None)
def _gather_kernel():
    return pl.kernel(
        _gather_body,
        out_type=[
            jax.ShapeDtypeStruct((EP, EH), jnp.float32),
            jax.ShapeDtypeStruct((EP, EH), jnp.float32),
        ],
        mesh=_sc_mesh(),
        compiler_params=pltpu.CompilerParams(use_tc_tiling_on_sc=False),
        scratch_types=[
            pltpu.VMEM((CPW, CH), jnp.int32),
            pltpu.VMEM((CPW, CH), jnp.int32),
            pltpu.VMEM((KF * CH, EH), jnp.float32),
            pltpu.VMEM((KF * CH, EH), jnp.float32),
            pltpu.SemaphoreType.DMA,
            pltpu.SemaphoreType.DMA,
            pltpu.SemaphoreType.DMA,
            pltpu.SemaphoreType.DMA,
        ],
    )


_EBLK = 1024
EP4 = EP // 4


def _edge_body(gs_ref, gd_ref, ea_ref, st_ref, SW_ref, BD2_ref, b2_ref,
               u_ref, un_ref):
    alpha = st_ref[0]
    beta = st_ref[1]
    c4t = ea_ref[...] * alpha + beta
    att = lax.dot_general(c4t, SW_ref[...], (((0,), (0,)), ((), ())),
                          preferred_element_type=jnp.float32)
    z = jax.nn.sigmoid(gs_ref[...] + gd_ref[...] + att)
    u = jax.nn.sigmoid(
        jnp.dot(z, BD2_ref[...], preferred_element_type=jnp.float32)
        + b2_ref[...])
    u_ref[...] = u
    un_ref[...] = -u


def _edge_call(gs4, gd4, ea4, stats, SW, BD2, b2t):
    g = EP4 // _EBLK
    return pl.pallas_call(
        _edge_body,
        grid=(g,),
        in_specs=[
            pl.BlockSpec((_EBLK, 128), lambda i: (i, 0)),
            pl.BlockSpec((_EBLK, 128), lambda i: (i, 0)),
            pl.BlockSpec((4, _EBLK), lambda i: (0, i)),
            pl.BlockSpec(memory_space=pltpu.SMEM),
            pl.BlockSpec((4, 128), lambda i: (0, 0)),
            pl.BlockSpec((128, 128), lambda i: (0, 0)),
            pl.BlockSpec((1, 128), lambda i: (0, 0)),
        ],
        out_specs=[
            pl.BlockSpec((_EBLK, 128), lambda i: (i, 0)),
            pl.BlockSpec((_EBLK, 128), lambda i: (i, 0)),
        ],
        out_shape=[
            jax.ShapeDtypeStruct((EP4, 128), jnp.float32),
            jax.ShapeDtypeStruct((EP4, 128), jnp.float32),
        ],
    )(gs4, gd4, ea4, stats, SW, BD2, b2t)


KF2 = 4
NPH2 = CPW // KF2


def _scatter_body(srcs, dsts, u, un, acc_out, idx_s, idx_d, ubuf, nbuf,
                  acc_p, sem_l, sem_l2, sem_sc, sem_sc2):
    c = lax.axis_index("c")
    s = lax.axis_index("s")
    w = s * NC + c

    zv = jnp.zeros((16,), jnp.float32)
    zrows = KF2 * CH

    def zstep(i, carry):
        ubuf[i, pl.ds(0, 16)] = zv
        ubuf[i, pl.ds(16, 16)] = zv
        nbuf[i, pl.ds(0, 16)] = zv
        nbuf[i, pl.ds(16, 16)] = zv
        return carry

    lax.fori_loop(0, zrows, zstep, 0)
    pltpu.sync_copy(ubuf, acc_p.at[pl.ds(s * RPT, zrows)])
    pltpu.sync_copy(nbuf, acc_p.at[pl.ds(s * RPT + zrows, zrows)])
    pltpu.sync_copy(ubuf.at[pl.ds(0, RPT - 2 * zrows)],
                    acc_p.at[pl.ds(s * RPT + 2 * zrows, RPT - 2 * zrows)])
    pltpu.sync_copy(srcs.at[w], idx_s)
    pltpu.sync_copy(dsts.at[w], idx_d)
    plsc.subcore_barrier()

    def do_chunks(base, nk):
        cps = []
        for k in range(nk):
            j = base + k
            b = pl.ds(k * CH, CH)
            o = pl.ds(w * EPW + j * CH, CH)
            cps.append(pltpu.async_copy(u.at[o], ubuf.at[b], sem_l))
            cps.append(pltpu.async_copy(un.at[o], nbuf.at[b], sem_l2))
        for cp in cps:
            cp.wait()
        sts = []
        for k in range(nk):
            j = base + k
            b = pl.ds(k * CH, CH)
            sts.append(pltpu.async_copy(ubuf.at[b], acc_p.at[idx_d.at[j]],
                                        sem_sc, add=True))
            sts.append(pltpu.async_copy(nbuf.at[b], acc_p.at[idx_s.at[j]],
                                        sem_sc2, add=True))
        for st in sts:
            st.wait()

    def phase(g, carry):
        do_chunks(g * KF2, KF2)
        return carry

    lax.fori_loop(0, NPH2, phase, 0)
    do_chunks(NPH2 * KF2, CPW - NPH2 * KF2)
    plsc.subcore_barrier()
    pltpu.sync_copy(acc_p.at[pl.ds(s * RPT, RPT)],
                    acc_out.at[c, pl.ds(s * RPT, RPT)])


@functools.lru_cache(maxsize=None)
def _scatter_kernel():
    return pl.kernel(
        _scatter_body,
        out_type=jax.ShapeDtypeStruct((NC, ACC_R, EOP), jnp.float32),
        mesh=_sc_mesh(),
        compiler_params=pltpu.CompilerParams(use_tc_tiling_on_sc=False),
        scratch_types=[
            pltpu.VMEM((CPW, CH), jnp.int32),
            pltpu.VMEM((CPW, CH), jnp.int32),
            pltpu.VMEM((KF2 * CH, EOP), jnp.float32),
            pltpu.VMEM((KF2 * CH, EOP), jnp.float32),
            pltpu.VMEM_SHARED((ACC_R, EOP), jnp.float32),
            pltpu.SemaphoreType.DMA,
            pltpu.SemaphoreType.DMA,
            pltpu.SemaphoreType.DMA,
            pltpu.SemaphoreType.DMA,
        ],
    )


_NBLK = 2000


def _final_body(p0_ref, p1_ref, W3_ref, b3_ref, o_ref):
    agg = p0_ref[...] + p1_ref[...]
    o_ref[...] = jax.nn.sigmoid(
        jnp.dot(agg, W3_ref[...], preferred_element_type=jnp.float32)
        + b3_ref[...])


def _final_call(p0, p1, W3p, b3r):
    g = BN // _NBLK
    return pl.pallas_call(
        _final_body,
        grid=(g,),
        in_specs=[
            pl.BlockSpec((_NBLK, EOP), lambda i: (i, 0)),
            pl.BlockSpec((_NBLK, EOP), lambda i: (i, 0)),
            pl.BlockSpec((EOP, NOUT), lambda i: (0, 0)),
            pl.BlockSpec((1, NOUT), lambda i: (0, 0)),
        ],
        out_specs=pl.BlockSpec((_NBLK, NOUT), lambda i: (i, 0)),
        out_shape=jax.ShapeDtypeStruct((BN, NOUT), jnp.float32),
    )(p0, p1, W3p, b3r)


def kernel(x, edge_index, edge_attr, W1, b1, W2, b2, W3, b3):
    f32 = jnp.float32
    x2 = x.reshape(BN, D).astype(f32)
    ea = edge_attr.astype(f32)
    src = edge_index[0].astype(jnp.int32)
    dst = edge_index[1].astype(jnp.int32)

    W1 = W1.astype(f32)
    W1a = W1[:D]
    W1b = W1[D:2 * D]
    w1c = W1[2 * D].reshape(1, EH)
    b1r = b1.astype(f32).reshape(1, EH)
    W2p = jnp.pad(W2.astype(f32), ((0, 0), (0, EOP - EO)))
    b2p = jnp.pad(b2.astype(f32), (0, EOP - EO)).reshape(1, EOP)
    W3p = jnp.pad(W3.astype(f32), ((0, EOP - EO), (0, 0)))
    b3r = b3.astype(f32).reshape(1, NOUT)

    npad = EP - B * E
    srcb = jnp.concatenate([src, src + N])
    dstb = jnp.concatenate([dst, dst + N])
    zpad = jnp.zeros((npad,), jnp.int32)
    dpad = jnp.full((npad,), BN, jnp.int32)
    srcg = jnp.concatenate([srcb, zpad]).reshape(NW, CPW, CH)
    dstg = jnp.concatenate([dstb, zpad]).reshape(NW, CPW, CH)
    srcs = jnp.concatenate([srcb, dpad]).reshape(NW, CPW, CH)
    dsts = jnp.concatenate([dstb, dpad]).reshape(NW, CPW, CH)
    ea2 = jnp.concatenate([ea, ea, jnp.zeros((npad, 1), f32)])

    eye4 = jnp.eye(4, dtype=f32)
    SW = jnp.kron(eye4, w1c)
    BD2 = jnp.kron(eye4, W2p)
    b2t = jnp.tile(b2p, (1, 4))
    eaf = ea2.reshape(EP)
    ea4 = jnp.stack([eaf[0::4], eaf[1::4], eaf[2::4], eaf[3::4]])

    stats = _stats_call(ea.reshape(E // 128, 128))
    hs, hd = _tables_call(x2, W1a, W1b, b1r)
    gs, gd = _gather_kernel()(srcg, dstg, hs, hd)
    gs4 = gs.reshape(EP // 4, 128)
    gd4 = gd.reshape(EP // 4, 128)
    u4, un4 = _edge_call(gs4, gd4, ea4, stats, SW, BD2, b2t)
    acc = _scatter_kernel()(srcs, dsts, u4.reshape(EP, EOP),
                            un4.reshape(EP, EOP))
    out = _final_call(acc[0, :BN], acc[1, :BN], W3p, b3r)
    return out.reshape(B, N, NOUT)


# speedup vs baseline: 114.4117x; 1.0130x over previous
import functools
import jax
import jax.numpy as jnp
from jax import lax
from jax.experimental import pallas as pl
from jax.experimental.pallas import tpu as pltpu
from jax.experimental.pallas import tpu_sc as plsc

N = 10000
E = 320000
D = 128
B = 2
EH = 32
EO = 30
EOP = 32
NOUT = 128

NC = 2
NS = 16
NW = NC * NS
CH = 128
CPW = (B * E + NW * CH - 1) // (NW * CH)
EPW = CPW * CH
EP = NW * EPW
BN = B * N
ACC_R = BN + NS
RPT = ACC_R // NS

@functools.lru_cache(maxsize=None)
def _sc_mesh():
    return plsc.VectorSubcoreMesh(
        core_axis_name="c", subcore_axis_name="s",
        num_cores=NC, num_subcores=NS)


def _stats_body(ea_ref, o_ref):
    ea = ea_ref[...]
    s = jnp.sum(ea)
    ss = jnp.sum(ea * ea)
    mean = s / E
    var = (ss - s * s / E) / (E - 1)
    alpha = lax.rsqrt(var)
    o_ref[0] = alpha
    o_ref[1] = -mean * alpha


def _stats_call(ea):
    return pl.pallas_call(
        _stats_body,
        out_specs=pl.BlockSpec(memory_space=pltpu.SMEM),
        out_shape=jax.ShapeDtypeStruct((2,), jnp.float32),
    )(ea)


_TBLK = 2000


def _tables_body(x_ref, wa_ref, wb_ref, b1_ref, hs_ref, hd_ref):
    xb = x_ref[...]
    hs_ref[...] = jnp.dot(xb, wa_ref[...],
                          preferred_element_type=jnp.float32) + b1_ref[...]
    hd_ref[...] = jnp.dot(xb, wb_ref[...],
                          preferred_element_type=jnp.float32)


def _tables_call(x2, W1a, W1b, b1r):
    g = BN // _TBLK
    return pl.pallas_call(
        _tables_body,
        grid=(g,),
        in_specs=[
            pl.BlockSpec((_TBLK, D), lambda i: (i, 0)),
            pl.BlockSpec((D, EH), lambda i: (0, 0)),
            pl.BlockSpec((D, EH), lambda i: (0, 0)),
            pl.BlockSpec((1, EH), lambda i: (0, 0)),
        ],
        out_specs=[
            pl.BlockSpec((_TBLK, EH), lambda i: (i, 0)),
            pl.BlockSpec((_TBLK, EH), lambda i: (i, 0)),
        ],
        out_shape=[
            jax.ShapeDtypeStruct((BN, EH), jnp.float32),
            jax.ShapeDtypeStruct((BN, EH), jnp.float32),
        ],
    )(x2, W1a, W1b, b1r)


KF = 8
NPH = CPW // KF


def _gather_body(srcg, dstg, hs, hd, gs, gd, idx_s, idx_d, buf_s, buf_d,
                 sem_s, sem_d, sem_ss, sem_sd):
    c = lax.axis_index("c")
    s = lax.axis_index("s")
    w = s * NC + c
    pltpu.sync_copy(srcg.at[w], idx_s)
    pltpu.sync_copy(dstg.at[w], idx_d)

    def do_chunks(base, nk):
        cps = []
        for k in range(nk):
            j = base + k
            b = pl.ds(k * CH, CH)
            cps.append(pltpu.async_copy(hs.at[idx_s.at[j]], buf_s.at[b],
                                        sem_s))
            cps.append(pltpu.async_copy(hd.at[idx_d.at[j]], buf_d.at[b],
                                        sem_d))
        for cp in cps:
            cp.wait()
        sts = []
        for k in range(nk):
            j = base + k
            b = pl.ds(k * CH, CH)
            o = pl.ds(w * EPW + j * CH, CH)
            sts.append(pltpu.async_copy(buf_s.at[b], gs.at[o], sem_ss))
            sts.append(pltpu.async_copy(buf_d.at[b], gd.at[o], sem_sd))
        for st in sts:
            st.wait()

    def phase(g, carry):
        do_chunks(g * KF, KF)
        return carry

    lax.fori_loop(0, NPH, phase, 0)
    do_chunks(NPH * KF, CPW - NPH * KF)


@functools.lru_cache(maxsize=None)
def _gather_kernel():
    return pl.kernel(
        _gather_body,
        out_type=[
            jax.ShapeDtypeStruct((EP, EH), jnp.float32),
            jax.ShapeDtypeStruct((EP, EH), jnp.float32),
        ],
        mesh=_sc_mesh(),
        compiler_params=pltpu.CompilerParams(use_tc_tiling_on_sc=False),
        scratch_types=[
            pltpu.VMEM((CPW, CH), jnp.int32),
            pltpu.VMEM((CPW, CH), jnp.int32),
            pltpu.VMEM((KF * CH, EH), jnp.float32),
            pltpu.VMEM((KF * CH, EH), jnp.float32),
            pltpu.SemaphoreType.DMA,
            pltpu.SemaphoreType.DMA,
            pltpu.SemaphoreType.DMA,
            pltpu.SemaphoreType.DMA,
        ],
    )


_EBLK = 1024
EP4 = EP // 4


def _edge_body(gs_ref, gd_ref, ea_ref, st_ref, SW_ref, BD2_ref, b2_ref,
               u_ref):
    alpha = st_ref[0]
    beta = st_ref[1]
    c4t = ea_ref[...] * alpha + beta
    att = lax.dot_general(c4t, SW_ref[...], (((0,), (0,)), ((), ())),
                          preferred_element_type=jnp.float32)
    z = jax.nn.sigmoid(gs_ref[...] + gd_ref[...] + att)
    u_ref[...] = jax.nn.sigmoid(
        jnp.dot(z, BD2_ref[...], preferred_element_type=jnp.float32)
        + b2_ref[...])


def _edge_call(gs4, gd4, ea4, stats, SW, BD2, b2t):
    g = EP4 // _EBLK
    return pl.pallas_call(
        _edge_body,
        grid=(g,),
        in_specs=[
            pl.BlockSpec((_EBLK, 128), lambda i: (i, 0)),
            pl.BlockSpec((_EBLK, 128), lambda i: (i, 0)),
            pl.BlockSpec((4, _EBLK), lambda i: (0, i)),
            pl.BlockSpec(memory_space=pltpu.SMEM),
            pl.BlockSpec((4, 128), lambda i: (0, 0)),
            pl.BlockSpec((128, 128), lambda i: (0, 0)),
            pl.BlockSpec((1, 128), lambda i: (0, 0)),
        ],
        out_specs=pl.BlockSpec((_EBLK, 128), lambda i: (i, 0)),
        out_shape=jax.ShapeDtypeStruct((EP4, 128), jnp.float32),
    )(gs4, gd4, ea4, stats, SW, BD2, b2t)


G2 = 32
KF2 = 8
NG2 = CPW // G2


def _scatter_body(srcs, dsts, u, acc_out, idx_s, idx_d, ubuf,
                  acc_p, acc_q, sem_l, sem_sc, sem_sc2):
    c = lax.axis_index("c")
    s = lax.axis_index("s")
    w = s * NC + c

    zv = jnp.zeros((16,), jnp.float32)
    zrows = KF2 * CH

    def zstep(i, carry):
        ubuf[i, pl.ds(0, 16)] = zv
        ubuf[i, pl.ds(16, 16)] = zv
        return carry

    lax.fori_loop(0, zrows, zstep, 0)
    for acc in (acc_p, acc_q):
        pltpu.sync_copy(ubuf, acc.at[pl.ds(s * RPT, zrows)])
        pltpu.sync_copy(ubuf.at[pl.ds(0, RPT - zrows)],
                        acc.at[pl.ds(s * RPT + zrows, RPT - zrows)])
    plsc.subcore_barrier()

    def do_phase(g0, l0, nk):
        cps = []
        for k in range(nk):
            b = pl.ds(k * CH, CH)
            o = pl.ds(w * EPW + (g0 + l0 + k) * CH, CH)
            cps.append(pltpu.async_copy(u.at[o], ubuf.at[b], sem_l))
        for cp in cps:
            cp.wait()
        sts = []
        for k in range(nk):
            b = pl.ds(k * CH, CH)
            sts.append(pltpu.async_copy(ubuf.at[b], acc_p.at[idx_d.at[l0 + k]],
                                        sem_sc, add=True))
            sts.append(pltpu.async_copy(ubuf.at[b], acc_q.at[idx_s.at[l0 + k]],
                                        sem_sc2, add=True))
        for st in sts:
            st.wait()

    def group(g0, nloc):
        pltpu.sync_copy(srcs.at[w, pl.ds(g0, nloc)], idx_s.at[pl.ds(0, nloc)])
        pltpu.sync_copy(dsts.at[w, pl.ds(g0, nloc)], idx_d.at[pl.ds(0, nloc)])
        for ph in range(nloc // KF2):
            do_phase(g0, ph * KF2, KF2)
        r = nloc % KF2
        if r:
            do_phase(g0, (nloc // KF2) * KF2, r)

    def grp_body(gi, carry):
        group(gi * G2, G2)
        return carry

    lax.fori_loop(0, NG2, grp_body, 0)
    group(NG2 * G2, CPW - NG2 * G2)
    plsc.subcore_barrier()
    pltpu.sync_copy(acc_p.at[pl.ds(s * RPT, RPT)],
                    acc_out.at[c, 0, pl.ds(s * RPT, RPT)])
    pltpu.sync_copy(acc_q.at[pl.ds(s * RPT, RPT)],
                    acc_out.at[c, 1, pl.ds(s * RPT, RPT)])


@functools.lru_cache(maxsize=None)
def _scatter_kernel():
    return pl.kernel(
        _scatter_body,
        out_type=jax.ShapeDtypeStruct((NC, 2, ACC_R, EOP), jnp.float32),
        mesh=_sc_mesh(),
        compiler_params=pltpu.CompilerParams(use_tc_tiling_on_sc=False),
        scratch_types=[
            pltpu.VMEM((G2, CH), jnp.int32),
            pltpu.VMEM((G2, CH), jnp.int32),
            pltpu.VMEM((KF2 * CH, EOP), jnp.float32),
            pltpu.VMEM_SHARED((ACC_R, EOP), jnp.float32),
            pltpu.VMEM_SHARED((ACC_R, EOP), jnp.float32),
            pltpu.SemaphoreType.DMA,
            pltpu.SemaphoreType.DMA,
            pltpu.SemaphoreType.DMA,
        ],
    )


_NBLK = 2000


def _final_body(p0_ref, p1_ref, q0_ref, q1_ref, W3_ref, b3_ref, o_ref):
    agg = (p0_ref[...] + p1_ref[...]) - (q0_ref[...] + q1_ref[...])
    o_ref[...] = jax.nn.sigmoid(
        jnp.dot(agg, W3_ref[...], preferred_element_type=jnp.float32)
        + b3_ref[...])


def _final_call(p0, p1, q0, q1, W3p, b3r):
    g = BN // _NBLK
    return pl.pallas_call(
        _final_body,
        grid=(g,),
        in_specs=[
            pl.BlockSpec((_NBLK, EOP), lambda i: (i, 0)),
            pl.BlockSpec((_NBLK, EOP), lambda i: (i, 0)),
            pl.BlockSpec((_NBLK, EOP), lambda i: (i, 0)),
            pl.BlockSpec((_NBLK, EOP), lambda i: (i, 0)),
            pl.BlockSpec((EOP, NOUT), lambda i: (0, 0)),
            pl.BlockSpec((1, NOUT), lambda i: (0, 0)),
        ],
        out_specs=pl.BlockSpec((_NBLK, NOUT), lambda i: (i, 0)),
        out_shape=jax.ShapeDtypeStruct((BN, NOUT), jnp.float32),
    )(p0, p1, q0, q1, W3p, b3r)


def kernel(x, edge_index, edge_attr, W1, b1, W2, b2, W3, b3):
    f32 = jnp.float32
    x2 = x.reshape(BN, D).astype(f32)
    ea = edge_attr.astype(f32)
    src = edge_index[0].astype(jnp.int32)
    dst = edge_index[1].astype(jnp.int32)

    W1 = W1.astype(f32)
    W1a = W1[:D]
    W1b = W1[D:2 * D]
    w1c = W1[2 * D].reshape(1, EH)
    b1r = b1.astype(f32).reshape(1, EH)
    W2p = jnp.pad(W2.astype(f32), ((0, 0), (0, EOP - EO)))
    b2p = jnp.pad(b2.astype(f32), (0, EOP - EO)).reshape(1, EOP)
    W3p = jnp.pad(W3.astype(f32), ((0, EOP - EO), (0, 0)))
    b3r = b3.astype(f32).reshape(1, NOUT)

    npad = EP - B * E
    srcb = jnp.concatenate([src, src + N])
    dstb = jnp.concatenate([dst, dst + N])
    zpad = jnp.zeros((npad,), jnp.int32)
    dpad = jnp.full((npad,), BN, jnp.int32)
    srcg = jnp.concatenate([srcb, zpad]).reshape(NW, CPW, CH)
    dstg = jnp.concatenate([dstb, zpad]).reshape(NW, CPW, CH)
    srcs = jnp.concatenate([srcb, dpad]).reshape(NW, CPW, CH)
    dsts = jnp.concatenate([dstb, dpad]).reshape(NW, CPW, CH)
    ea2 = jnp.concatenate([ea, ea, jnp.zeros((npad, 1), f32)])

    eye4 = jnp.eye(4, dtype=f32)
    SW = jnp.kron(eye4, w1c)
    BD2 = jnp.kron(eye4, W2p)
    b2t = jnp.tile(b2p, (1, 4))
    eaf = ea2.reshape(EP)
    ea4 = jnp.stack([eaf[0::4], eaf[1::4], eaf[2::4], eaf[3::4]])

    stats = _stats_call(ea.reshape(E // 128, 128))
    hs, hd = _tables_call(x2, W1a, W1b, b1r)
    gs, gd = _gather_kernel()(srcg, dstg, hs, hd)
    gs4 = gs.reshape(EP // 4, 128)
    gd4 = gd.reshape(EP // 4, 128)
    u4 = _edge_call(gs4, gd4, ea4, stats, SW, BD2, b2t)
    acc = _scatter_kernel()(srcs, dsts, u4.reshape(EP, EOP))
    out = _final_call(acc[0, 0, :BN], acc[1, 0, :BN],
                      acc[0, 1, :BN], acc[1, 1, :BN], W3p, b3r)
    return out.reshape(B, N, NOUT)


# speedup vs baseline: 119.2460x; 1.0423x over previous
import functools
import jax
import jax.numpy as jnp
from jax import lax
from jax.experimental import pallas as pl
from jax.experimental.pallas import tpu as pltpu
from jax.experimental.pallas import tpu_sc as plsc

N = 10000
E = 320000
D = 128
B = 2
EH = 32
EO = 30
EOP = 32
NOUT = 128

NC = 2
NS = 16
NW = NC * NS
CH = 128
CPB = (E + NW * CH - 1) // (NW * CH)
EPW = CPB * CH
EPB = NW * EPW
EPB4 = EPB // 4
BN = B * N
ACC_R = N + NS
RPT = ACC_R // NS

KF = 8
G2 = 32
KF2 = 8


@functools.lru_cache(maxsize=None)
def _sc_mesh():
    return plsc.VectorSubcoreMesh(
        core_axis_name="c", subcore_axis_name="s",
        num_cores=NC, num_subcores=NS)


def _stats_body(ea_ref, o_ref):
    ea = ea_ref[...]
    s = jnp.sum(ea)
    ss = jnp.sum(ea * ea)
    mean = s / E
    var = (ss - s * s / E) / (E - 1)
    alpha = lax.rsqrt(var)
    o_ref[0] = alpha
    o_ref[1] = -mean * alpha


def _stats_call(ea):
    return pl.pallas_call(
        _stats_body,
        out_specs=pl.BlockSpec(memory_space=pltpu.SMEM),
        out_shape=jax.ShapeDtypeStruct((2,), jnp.float32),
    )(ea)


_TBLK = 2000


def _tables_body(x_ref, wa_ref, wb_ref, b1_ref, hs_ref, hd_ref):
    xb = x_ref[...]
    hs_ref[...] = jnp.dot(xb, wa_ref[...],
                          preferred_element_type=jnp.float32) + b1_ref[...]
    hd_ref[...] = jnp.dot(xb, wb_ref[...],
                          preferred_element_type=jnp.float32)


def _tables_call(x2, W1a, W1b, b1r):
    g = BN // _TBLK
    return pl.pallas_call(
        _tables_body,
        grid=(g,),
        in_specs=[
            pl.BlockSpec((_TBLK, D), lambda i: (i, 0)),
            pl.BlockSpec((D, EH), lambda i: (0, 0)),
            pl.BlockSpec((D, EH), lambda i: (0, 0)),
            pl.BlockSpec((1, EH), lambda i: (0, 0)),
        ],
        out_specs=[
            pl.BlockSpec((_TBLK, EH), lambda i: (i, 0)),
            pl.BlockSpec((_TBLK, EH), lambda i: (i, 0)),
        ],
        out_shape=[
            jax.ShapeDtypeStruct((BN, EH), jnp.float32),
            jax.ShapeDtypeStruct((BN, EH), jnp.float32),
        ],
    )(x2, W1a, W1b, b1r)


def _gather_body(srcg, dstg, hs, hd, gs, gd, idx_s, idx_d, buf_s, buf_d,
                 sem_s, sem_d, sem_ss, sem_sd):
    c = lax.axis_index("c")
    s = lax.axis_index("s")
    w = s * NC + c
    pltpu.sync_copy(srcg.at[w], idx_s)
    pltpu.sync_copy(dstg.at[w], idx_d)

    def do_chunks(base, nk):
        cps = []
        for k in range(nk):
            j = base + k
            b = pl.ds(k * CH, CH)
            cps.append(pltpu.async_copy(hs.at[idx_s.at[j]], buf_s.at[b],
                                        sem_s))
            cps.append(pltpu.async_copy(hd.at[idx_d.at[j]], buf_d.at[b],
                                        sem_d))
        for cp in cps:
            cp.wait()
        sts = []
        for k in range(nk):
            j = base + k
            b = pl.ds(k * CH, CH)
            o = pl.ds(w * EPW + j * CH, CH)
            sts.append(pltpu.async_copy(buf_s.at[b], gs.at[o], sem_ss))
            sts.append(pltpu.async_copy(buf_d.at[b], gd.at[o], sem_sd))
        for st in sts:
            st.wait()

    nph = CPB // KF

    def phase(g, carry):
        do_chunks(g * KF, KF)
        return carry

    lax.fori_loop(0, nph, phase, 0)
    do_chunks(nph * KF, CPB - nph * KF)


@functools.lru_cache(maxsize=None)
def _gather_kernel():
    return pl.kernel(
        _gather_body,
        out_type=[
            jax.ShapeDtypeStruct((EPB, EH), jnp.float32),
            jax.ShapeDtypeStruct((EPB, EH), jnp.float32),
        ],
        mesh=_sc_mesh(),
        compiler_params=pltpu.CompilerParams(use_tc_tiling_on_sc=False),
        scratch_types=[
            pltpu.VMEM((CPB, CH), jnp.int32),
            pltpu.VMEM((CPB, CH), jnp.int32),
            pltpu.VMEM((KF * CH, EH), jnp.float32),
            pltpu.VMEM((KF * CH, EH), jnp.float32),
            pltpu.SemaphoreType.DMA,
            pltpu.SemaphoreType.DMA,
            pltpu.SemaphoreType.DMA,
            pltpu.SemaphoreType.DMA,
        ],
    )


_EBLK = 1024


def _edge_body(gs_ref, gd_ref, ea_ref, st_ref, SW_ref, BD2_ref, b2_ref,
               u_ref):
    alpha = st_ref[0]
    beta = st_ref[1]
    c4t = ea_ref[...] * alpha + beta
    att = lax.dot_general(c4t, SW_ref[...], (((0,), (0,)), ((), ())),
                          preferred_element_type=jnp.float32)
    z = jax.nn.sigmoid(gs_ref[...] + gd_ref[...] + att)
    u_ref[...] = jax.nn.sigmoid(
        jnp.dot(z, BD2_ref[...], preferred_element_type=jnp.float32)
        + b2_ref[...])


def _edge_call(gs4, gd4, ea4, stats, SW, BD2, b2t):
    g = EPB4 // _EBLK
    return pl.pallas_call(
        _edge_body,
        grid=(g,),
        in_specs=[
            pl.BlockSpec((_EBLK, 128), lambda i: (i, 0)),
            pl.BlockSpec((_EBLK, 128), lambda i: (i, 0)),
            pl.BlockSpec((4, _EBLK), lambda i: (0, i)),
            pl.BlockSpec(memory_space=pltpu.SMEM),
            pl.BlockSpec((4, 128), lambda i: (0, 0)),
            pl.BlockSpec((128, 128), lambda i: (0, 0)),
            pl.BlockSpec((1, 128), lambda i: (0, 0)),
        ],
        out_specs=pl.BlockSpec((_EBLK, 128), lambda i: (i, 0)),
        out_shape=jax.ShapeDtypeStruct((EPB4, 128), jnp.float32),
    )(gs4, gd4, ea4, stats, SW, BD2, b2t)


def _scatter_body(srcs, dsts, u, acc_out, idx_s, idx_d, ubuf,
                  acc_p, acc_q, sem_l, sem_sc, sem_sc2):
    c = lax.axis_index("c")
    s = lax.axis_index("s")
    w = s * NC + c

    zv = jnp.zeros((16,), jnp.float32)

    def zstep(i, carry):
        ubuf[i, pl.ds(0, 16)] = zv
        ubuf[i, pl.ds(16, 16)] = zv
        return carry

    lax.fori_loop(0, RPT, zstep, 0)
    pltpu.sync_copy(ubuf.at[pl.ds(0, RPT)], acc_p.at[pl.ds(s * RPT, RPT)])
    pltpu.sync_copy(ubuf.at[pl.ds(0, RPT)], acc_q.at[pl.ds(s * RPT, RPT)])
    plsc.subcore_barrier()

    def do_phase(g0, l0, nk):
        cps = []
        for k in range(nk):
            b = pl.ds(k * CH, CH)
            o = pl.ds(w * EPW + (g0 + l0 + k) * CH, CH)
            cps.append(pltpu.async_copy(u.at[o], ubuf.at[b], sem_l))
        for cp in cps:
            cp.wait()
        sts = []
        for k in range(nk):
            b = pl.ds(k * CH, CH)
            sts.append(pltpu.async_copy(ubuf.at[b], acc_p.at[idx_d.at[l0 + k]],
                                        sem_sc, add=True))
            sts.append(pltpu.async_copy(ubuf.at[b], acc_q.at[idx_s.at[l0 + k]],
                                        sem_sc2, add=True))
        for st in sts:
            st.wait()

    def group(g0, nloc):
        pltpu.sync_copy(srcs.at[w, pl.ds(g0, nloc)], idx_s.at[pl.ds(0, nloc)])
        pltpu.sync_copy(dsts.at[w, pl.ds(g0, nloc)], idx_d.at[pl.ds(0, nloc)])
        for ph in range(nloc // KF2):
            do_phase(g0, ph * KF2, KF2)
        r = nloc % KF2
        if r:
            do_phase(g0, (nloc // KF2) * KF2, r)

    ng = CPB // G2

    def grp_body(gi, carry):
        group(gi * G2, G2)
        return carry

    lax.fori_loop(0, ng, grp_body, 0)
    group(ng * G2, CPB - ng * G2)
    plsc.subcore_barrier()
    pltpu.sync_copy(acc_p.at[pl.ds(s * RPT, RPT)],
                    acc_out.at[c, 0, pl.ds(s * RPT, RPT)])
    pltpu.sync_copy(acc_q.at[pl.ds(s * RPT, RPT)],
                    acc_out.at[c, 1, pl.ds(s * RPT, RPT)])


@functools.lru_cache(maxsize=None)
def _scatter_kernel():
    return pl.kernel(
        _scatter_body,
        out_type=jax.ShapeDtypeStruct((NC, 2, ACC_R, EOP), jnp.float32),
        mesh=_sc_mesh(),
        compiler_params=pltpu.CompilerParams(use_tc_tiling_on_sc=False),
        scratch_types=[
            pltpu.VMEM((G2, CH), jnp.int32),
            pltpu.VMEM((G2, CH), jnp.int32),
            pltpu.VMEM((KF2 * CH, EOP), jnp.float32),
            pltpu.VMEM_SHARED((ACC_R, EOP), jnp.float32),
            pltpu.VMEM_SHARED((ACC_R, EOP), jnp.float32),
            pltpu.SemaphoreType.DMA,
            pltpu.SemaphoreType.DMA,
            pltpu.SemaphoreType.DMA,
        ],
    )


_NBLK = 2000


def _final_body(p0_ref, p1_ref, q0_ref, q1_ref, W3_ref, b3_ref, o_ref):
    agg = (p0_ref[...] + p1_ref[...]) - (q0_ref[...] + q1_ref[...])
    o_ref[...] = jax.nn.sigmoid(
        jnp.dot(agg, W3_ref[...], preferred_element_type=jnp.float32)
        + b3_ref[...])


def _final_call(p0, p1, q0, q1, W3p, b3r):
    g = BN // _NBLK
    return pl.pallas_call(
        _final_body,
        grid=(g,),
        in_specs=[
            pl.BlockSpec((_NBLK, EOP), lambda i: (i, 0)),
            pl.BlockSpec((_NBLK, EOP), lambda i: (i, 0)),
            pl.BlockSpec((_NBLK, EOP), lambda i: (i, 0)),
            pl.BlockSpec((_NBLK, EOP), lambda i: (i, 0)),
            pl.BlockSpec((EOP, NOUT), lambda i: (0, 0)),
            pl.BlockSpec((1, NOUT), lambda i: (0, 0)),
        ],
        out_specs=pl.BlockSpec((_NBLK, NOUT), lambda i: (i, 0)),
        out_shape=jax.ShapeDtypeStruct((BN, NOUT), jnp.float32),
    )(p0, p1, q0, q1, W3p, b3r)


def kernel(x, edge_index, edge_attr, W1, b1, W2, b2, W3, b3):
    f32 = jnp.float32
    x2 = x.reshape(BN, D).astype(f32)
    ea = edge_attr.astype(f32)
    src = edge_index[0].astype(jnp.int32)
    dst = edge_index[1].astype(jnp.int32)

    W1 = W1.astype(f32)
    W1a = W1[:D]
    W1b = W1[D:2 * D]
    w1c = W1[2 * D].reshape(1, EH)
    b1r = b1.astype(f32).reshape(1, EH)
    W2p = jnp.pad(W2.astype(f32), ((0, 0), (0, EOP - EO)))
    b2p = jnp.pad(b2.astype(f32), (0, EOP - EO)).reshape(1, EOP)
    W3p = jnp.pad(W3.astype(f32), ((0, EOP - EO), (0, 0)))
    b3r = b3.astype(f32).reshape(1, NOUT)

    eye4 = jnp.eye(4, dtype=f32)
    SW = jnp.kron(eye4, w1c)
    BD2 = jnp.kron(eye4, W2p)
    b2t = jnp.tile(b2p, (1, 4))

    npad = EPB - E
    zpad = jnp.zeros((npad,), jnp.int32)
    dpad = jnp.full((npad,), N, jnp.int32)
    srcg = [jnp.concatenate([src + b * N, zpad]).reshape(NW, CPB, CH)
            for b in range(B)]
    dstg = [jnp.concatenate([dst + b * N, zpad]).reshape(NW, CPB, CH)
            for b in range(B)]
    srcs = jnp.concatenate([src, dpad]).reshape(NW, CPB, CH)
    dsts = jnp.concatenate([dst, dpad]).reshape(NW, CPB, CH)
    eaf = jnp.concatenate([ea.reshape(E), jnp.zeros((npad,), f32)])
    ea4 = jnp.stack([eaf[0::4], eaf[1::4], eaf[2::4], eaf[3::4]])

    stats = _stats_call(ea.reshape(E // 128, 128))
    hs, hd = _tables_call(x2, W1a, W1b, b1r)

    accs = []
    for b in range(B):
        gs, gd = _gather_kernel()(srcg[b], dstg[b], hs, hd)
        u4 = _edge_call(gs.reshape(EPB4, 128), gd.reshape(EPB4, 128),
                        ea4, stats, SW, BD2, b2t)
        accs.append(_scatter_kernel()(srcs, dsts, u4.reshape(EPB, EOP)))

    def cat(ci, pi):
        return jnp.concatenate([accs[0][ci, pi, :N], accs[1][ci, pi, :N]])

    out = _final_call(cat(0, 0), cat(1, 0), cat(0, 1), cat(1, 1), W3p, b3r)
    return out.reshape(B, N, NOUT)
